# baseline (speedup 1.0000x reference)
import functools

import jax
import jax.numpy as jnp
from jax import lax
from jax.experimental import pallas as pl
from jax.experimental.pallas import tpu as pltpu
from jax.experimental.pallas import tpu_sc as plsc

MAXLEN = 200
EMBED_DIM = 32
LANES = 16
NC = 2
NS = 16
NW = NC * NS

BT = 128
SG = 4
CB = SG * BT
DT = EMBED_DIM // 8
PITCH = BT + 1


@jax.jit
def _run(token_table, x, pos_table):
    Bb, S = x.shape
    n_per_w = S // SG
    assert n_per_w % 2 == 0 and Bb // BT == NW
    mesh = plsc.VectorSubcoreMesh(
        core_axis_name="c", subcore_axis_name="s", num_cores=NC, num_subcores=NS
    )

    @functools.partial(
        pl.kernel,
        out_type=jax.ShapeDtypeStruct((S, DT, Bb // BT, 8, BT), jnp.float32),
        mesh=mesh,
        scratch_types=[
            pltpu.VMEM((BT, MAXLEN), jnp.int32),
            pltpu.VMEM((2, SG, BT), jnp.int32),
            pltpu.VMEM((2, SG, BT, EMBED_DIM), jnp.float32),
            pltpu.VMEM((2, SG, DT, 8, PITCH), jnp.float32),
            pltpu.VMEM((MAXLEN, EMBED_DIM), jnp.float32),
            pltpu.SemaphoreType.DMA,
            pltpu.SemaphoreType.DMA,
            pltpu.SemaphoreType.DMA,
            pltpu.SemaphoreType.DMA,
        ],
        compiler_params=pltpu.CompilerParams(
            use_tc_tiling_on_sc=False, needs_layout_passes=False
        ),
    )
    def k(tok_hbm, x_hbm, pos_hbm, out_hbm, xst_v, idxT_v, grows_v, obuf_v,
          pos_v, g0, g1, s0, s1):
        gsem = (g0, g1)
        ssem = (s0, s1)
        wid = lax.axis_index("s") * NC + lax.axis_index("c")
        pltpu.sync_copy(pos_hbm, pos_v)
        pltpu.sync_copy(x_hbm.at[pl.ds(wid * BT, BT)], xst_v)

        d16 = lax.broadcasted_iota(jnp.int32, (LANES,), 0)
        i_dt = lax.shift_right_logical(d16, 3)
        i_dr = lax.bitwise_and(d16, 7)
        v_rows = [d16 + blk * LANES for blk in range(BT // LANES)]

        def build_idx(cloc, buf):
            for j in range(SG):
                i_col = jnp.full((LANES,), 0, jnp.int32) + (cloc * SG + j)
                for blk in range(BT // LANES):
                    idxT_v[buf, j, pl.ds(blk * LANES, LANES)] = (
                        plsc.load_gather(xst_v, [v_rows[blk], i_col])
                    )

        def fire_gathers(buf):
            for j in range(SG):
                pltpu.async_copy(
                    tok_hbm.at[idxT_v.at[buf, j]],
                    grows_v.at[buf, j],
                    gsem[buf],
                )

        def drain_gathers(buf):
            for j in range(SG):
                pltpu.make_async_copy(
                    tok_hbm.at[idxT_v.at[buf, j]],
                    grows_v.at[buf, j],
                    gsem[buf],
                ).wait()

        def process(cloc, buf):
            s0_ = cloc * SG

            for j in range(SG):
                pos0 = pos_v[s0_ + j, pl.ds(0, LANES)]
                pos1 = pos_v[s0_ + j, pl.ds(LANES, LANES)]
                ob3 = obuf_v.at[buf, j]

                def tr_body(b16, carry, j=j, ob3=ob3, pos0=pos0, pos1=pos1):
                    tl = b16 * LANES
                    for u in range(LANES):
                        t = tl + u
                        i_br = jnp.full((LANES,), 0, jnp.int32) + t
                        r0 = grows_v[buf, j, t, pl.ds(0, LANES)] + pos0
                        r1 = grows_v[buf, j, t, pl.ds(LANES, LANES)] + pos1
                        plsc.store_scatter(ob3, [i_dt, i_dr, i_br], r0)
                        plsc.store_scatter(ob3, [i_dt + 2, i_dr, i_br], r1)
                    return carry

                lax.fori_loop(0, BT // LANES, tr_body, 0)

            for j in range(SG):
                for dt in range(DT):
                    pltpu.async_copy(
                        obuf_v.at[buf, j, dt, :, pl.ds(0, BT)],
                        out_hbm.at[s0_ + j, dt, wid],
                        ssem[buf],
                    )

        def wait_out(buf):
            for j in range(SG):
                for dt in range(DT):
                    pltpu.make_async_copy(
                        obuf_v.at[buf, j, dt, :, pl.ds(0, BT)],
                        out_hbm.at[j, dt, 0],
                        ssem[buf],
                    ).wait()

        build_idx(0, 0)
        fire_gathers(0)

        def pair_body(i, carry):
            for b in range(2):
                cloc = i * 2 + b
                nb = 1 - b

                @pl.when(cloc + 1 < n_per_w)
                def _prefetch():
                    build_idx(cloc + 1, nb)
                    fire_gathers(nb)

                drain_gathers(b)

                @pl.when(cloc >= 2)
                def _reuse():
                    wait_out(b)

                process(cloc, b)
            return carry

        lax.fori_loop(0, n_per_w // 2, pair_body, 0)
        wait_out(0)
        wait_out(1)

    return k(token_table, x, pos_table)


def kernel(x, token_table, pos_table):
    B_rows, S = x.shape
    out5 = _run(token_table, x.astype(jnp.int32), pos_table)
    return out5.transpose(2, 4, 0, 1, 3).reshape(B_rows, S, EMBED_DIM)


# speedup vs baseline: 1.0251x; 1.0251x over previous
import functools

import jax
import jax.numpy as jnp
from jax import lax
from jax.experimental import pallas as pl
from jax.experimental.pallas import tpu as pltpu
from jax.experimental.pallas import tpu_sc as plsc

MAXLEN = 200
EMBED_DIM = 32
LANES = 16
NC = 2
NS = 16
NW = NC * NS

BT = 128
NBT = 4
CB = NBT * BT
DT = EMBED_DIM // 8
PITCH = BT + 1


@jax.jit
def _run(token_table, xT, pos_table):
    S, Bb = xT.shape
    n_per_w = (S * Bb // CB) // NW
    assert n_per_w % 2 == 0
    g_per_s = Bb // CB
    mesh = plsc.VectorSubcoreMesh(
        core_axis_name="c", subcore_axis_name="s", num_cores=NC, num_subcores=NS
    )

    @functools.partial(
        pl.kernel,
        out_type=jax.ShapeDtypeStruct((S, DT, Bb // BT, 8, BT), jnp.float32),
        mesh=mesh,
        scratch_types=[
            pltpu.VMEM((2, CB), jnp.int32),
            pltpu.VMEM((2, NBT, BT, EMBED_DIM), jnp.float32),
            pltpu.VMEM((2, NBT, DT, 8, PITCH), jnp.float32),
            pltpu.VMEM((MAXLEN, EMBED_DIM), jnp.float32),
            pltpu.SemaphoreType.DMA,
            pltpu.SemaphoreType.DMA,
            pltpu.SemaphoreType.DMA,
            pltpu.SemaphoreType.DMA,
            pltpu.SemaphoreType.DMA,
            pltpu.SemaphoreType.DMA,
        ],
        compiler_params=pltpu.CompilerParams(
            use_tc_tiling_on_sc=False, needs_layout_passes=False
        ),
    )
    def k(tok_hbm, x_hbm, pos_hbm, out_hbm, idx_v, grows_v, obuf_v, pos_v,
          g0, g1, s0, s1, i0, i1):
        gsem = (g0, g1)
        ssem = (s0, s1)
        isem = (i0, i1)
        wid = lax.axis_index("s") * NC + lax.axis_index("c")
        c0 = wid * n_per_w
        pltpu.sync_copy(pos_hbm, pos_v)

        d16 = lax.broadcasted_iota(jnp.int32, (LANES,), 0)
        i_dt = lax.shift_right_logical(d16, 3)
        i_dr = lax.bitwise_and(d16, 7)

        def idx_start(cloc, buf):
            c = c0 + cloc
            s = c // g_per_s
            b0 = (c % g_per_s) * CB
            pltpu.async_copy(
                x_hbm.at[s, pl.ds(b0, CB)], idx_v.at[buf], isem[buf]
            )

        def idx_wait(buf):
            pltpu.make_async_copy(
                x_hbm.at[0, pl.ds(0, CB)], idx_v.at[buf], isem[buf]
            ).wait()

        def fire_gathers(buf):
            for j in range(NBT):
                pltpu.async_copy(
                    tok_hbm.at[idx_v.at[buf, pl.ds(j * BT, BT)]],
                    grows_v.at[buf, j],
                    gsem[buf],
                )

        def drain_gathers(buf):
            for j in range(NBT):
                pltpu.make_async_copy(
                    tok_hbm.at[idx_v.at[buf, pl.ds(j * BT, BT)]],
                    grows_v.at[buf, j],
                    gsem[buf],
                ).wait()

        def process(cloc, buf):
            c = c0 + cloc
            s = c // g_per_s
            bt0 = (c % g_per_s) * NBT
            pos0 = pos_v[s, pl.ds(0, LANES)]
            pos1 = pos_v[s, pl.ds(LANES, LANES)]

            for btc in range(NBT):
                ob3 = obuf_v.at[buf, btc]

                def tr_body(b16, carry, btc=btc, ob3=ob3):
                    tl = b16 * LANES
                    for u in range(LANES):
                        t = tl + u
                        i_br = jnp.full((LANES,), 0, jnp.int32) + t
                        r0 = grows_v[buf, btc, t, pl.ds(0, LANES)] + pos0
                        r1 = grows_v[buf, btc, t, pl.ds(LANES, LANES)] + pos1
                        plsc.store_scatter(ob3, [i_dt, i_dr, i_br], r0)
                        plsc.store_scatter(ob3, [i_dt + 2, i_dr, i_br], r1)
                    return carry

                lax.fori_loop(0, BT // LANES, tr_body, 0)

            for btc in range(NBT):
                for dt in range(DT):
                    pltpu.async_copy(
                        obuf_v.at[buf, btc, dt, :, pl.ds(0, BT)],
                        out_hbm.at[s, dt, bt0 + btc],
                        ssem[buf],
                    )

        def wait_out(buf):
            for btc in range(NBT):
                for dt in range(DT):
                    pltpu.make_async_copy(
                        obuf_v.at[buf, btc, dt, :, pl.ds(0, BT)],
                        out_hbm.at[0, dt, btc],
                        ssem[buf],
                    ).wait()

        idx_start(0, 0)
        idx_wait(0)
        fire_gathers(0)
        idx_start(1, 1)

        def pair_body(i, carry):
            for b in range(2):
                cloc = i * 2 + b
                nb = 1 - b
                drain_gathers(b)

                @pl.when(cloc + 2 < n_per_w)
                def _pref_idx():
                    idx_start(cloc + 2, b)

                @pl.when(cloc + 1 < n_per_w)
                def _pref_gather():
                    idx_wait(nb)
                    fire_gathers(nb)

                @pl.when(cloc >= 2)
                def _reuse():
                    wait_out(b)

                process(cloc, b)
            return carry

        lax.fori_loop(0, n_per_w // 2, pair_body, 0)
        wait_out(0)
        wait_out(1)

    return k(token_table, xT, pos_table)


def kernel(x, token_table, pos_table):
    B_rows, S = x.shape
    out5 = _run(token_table, x.T.astype(jnp.int32), pos_table)
    return out5.transpose(2, 4, 0, 1, 3).reshape(B_rows, S, EMBED_DIM)
